# baseline (speedup 1.0000x reference)
import jax
import jax.numpy as jnp
from jax import lax
from jax.experimental import pallas as pl
from jax.experimental.pallas import tpu as pltpu
from jax.experimental.pallas import tpu_sc as plsc

N = 10000
E = 320000
D = 128
H = 128
OUT = 16

NC = 2
NS = 16
EPW = E // (NC * NS)
CH = 80
NCHUNK = EPW // CH
RPW = 624
EXTRA_ROW0 = NS * RPW
RCH = 208
NRCH = RPW // RCH


def _segsum_body(m_hbm, src_hbm, dst_hbm, parts_hbm,
                 acc, src0, dst0, src1, dst1, src2, dst2, src3, dst3,
                 rows0, rows1, zbuf,
                 semi0, semi1, semi2, semi3, semg0, semg1, sems0, sems1):
    c = lax.axis_index("c")
    s = lax.axis_index("s")
    row0 = s * RPW
    erow = EXTRA_ROW0 + s * 8
    SRC = (src0, src1, src2, src3)
    DST = (dst0, dst1, dst2, dst3)
    ROWS = (rows0, rows1)
    SEMI = (semi0, semi1, semi2, semi3)
    SEMG = (semg0, semg1)
    SEMS = (sems0, sems1)

    def zrow(r, carry):
        for u in range(8):
            zbuf[r, pl.ds(u * 16, 16)] = jnp.zeros((16,), jnp.float32)
        return carry
    lax.fori_loop(0, RCH, zrow, 0)

    def zacc(j, carry):
        pltpu.sync_copy(zbuf, acc.at[pl.ds(row0 + j * RCH, RCH)])
        return carry
    lax.fori_loop(0, NRCH, zacc, 0)

    @pl.when(s < 2)
    def _():
        pltpu.sync_copy(zbuf.at[pl.ds(0, 8)], acc.at[pl.ds(erow, 8)])
    plsc.subcore_barrier()

    base0 = (c * NS + s) * EPW

    def issue_idx(j, q):
        b = base0 + jnp.minimum(j, NCHUNK - 1) * CH
        pltpu.async_copy(src_hbm.at[pl.ds(b, CH)], SRC[q], SEMI[q])
        pltpu.async_copy(dst_hbm.at[pl.ds(b, CH)], DST[q], SEMI[q])

    def wait_idx(q):
        pltpu.make_async_copy(src_hbm.at[pl.ds(0, CH)], SRC[q], SEMI[q]).wait()
        pltpu.make_async_copy(dst_hbm.at[pl.ds(0, CH)], DST[q], SEMI[q]).wait()

    def issue_gather(p, q):
        pltpu.async_copy(m_hbm.at[SRC[q]], ROWS[p], SEMG[p])

    def wait_gather(p, q):
        pltpu.make_async_copy(m_hbm.at[SRC[q]], ROWS[p], SEMG[p]).wait()

    def issue_scatter(p, q):
        pltpu.async_copy(ROWS[p], acc.at[DST[q]], SEMS[p], add=True)

    def wait_scatter(p, q):
        pltpu.make_async_copy(ROWS[p], acc.at[DST[q]], SEMS[p]).wait()

    def hstep(j, q, p):
        wait_idx((q + 1) % 4)
        wait_scatter(1 - p, (q + 3) % 4)
        issue_gather(1 - p, (q + 1) % 4)
        issue_idx(j + 3, (q + 3) % 4)
        wait_gather(p, q)
        issue_scatter(p, q)

    issue_idx(0, 0)
    issue_idx(1, 1)
    issue_idx(2, 2)
    wait_idx(0)
    issue_gather(0, 0)
    wait_idx(1)
    issue_gather(1, 1)
    issue_idx(3, 3)
    wait_gather(0, 0)
    issue_scatter(0, 0)

    def step4(t, carry):
        j = 4 * t + 1
        hstep(j, 1, 1)
        hstep(j + 1, 2, 0)
        hstep(j + 2, 3, 1)
        hstep(j + 3, 0, 0)
        return carry
    lax.fori_loop(0, (NCHUNK - 1) // 4, step4, 0)

    wait_idx(2)
    wait_idx(3)
    wait_gather(1, 1)
    wait_scatter(0, 0)
    plsc.subcore_barrier()

    pltpu.sync_copy(acc.at[pl.ds(row0, RPW)], parts_hbm.at[c, pl.ds(row0, RPW)])

    @pl.when(s < 2)
    def _():
        pltpu.sync_copy(acc.at[pl.ds(erow, 8)], parts_hbm.at[c, pl.ds(erow, 8)])


_segsum = pl.kernel(
    _segsum_body,
    out_type=jax.ShapeDtypeStruct((NC, N, H), jnp.float32),
    mesh=plsc.VectorSubcoreMesh(core_axis_name="c", subcore_axis_name="s",
                                num_cores=NC, num_subcores=NS),
    scratch_types=[
        pltpu.VMEM_SHARED((N, H), jnp.float32),
        pltpu.VMEM((CH,), jnp.int32),
        pltpu.VMEM((CH,), jnp.int32),
        pltpu.VMEM((CH,), jnp.int32),
        pltpu.VMEM((CH,), jnp.int32),
        pltpu.VMEM((CH,), jnp.int32),
        pltpu.VMEM((CH,), jnp.int32),
        pltpu.VMEM((CH,), jnp.int32),
        pltpu.VMEM((CH,), jnp.int32),
        pltpu.VMEM((CH, H), jnp.float32),
        pltpu.VMEM((CH, H), jnp.float32),
        pltpu.VMEM((RCH, H), jnp.float32),
        pltpu.SemaphoreType.DMA,
        pltpu.SemaphoreType.DMA,
        pltpu.SemaphoreType.DMA,
        pltpu.SemaphoreType.DMA,
        pltpu.SemaphoreType.DMA,
        pltpu.SemaphoreType.DMA,
        pltpu.SemaphoreType.DMA,
        pltpu.SemaphoreType.DMA,
    ],
)


BLK = 2000


def _layer_body(p_ref, x_ref, wr_ref, wo_ref, b_ref, h_ref):
    pp = p_ref[...]
    agg = pp[0] + pp[1]
    h_ref[...] = jnp.maximum(
        jnp.dot(agg, wr_ref[...], preferred_element_type=jnp.float32)
        + b_ref[...]
        + jnp.dot(x_ref[...], wo_ref[...], preferred_element_type=jnp.float32),
        0.0)


_layer = pl.pallas_call(
    _layer_body,
    grid=(N // BLK,),
    in_specs=[
        pl.BlockSpec((NC, BLK, H), lambda i: (0, i, 0)),
        pl.BlockSpec((BLK, H), lambda i: (i, 0)),
        pl.BlockSpec((H, H), lambda i: (0, 0)),
        pl.BlockSpec((H, H), lambda i: (0, 0)),
        pl.BlockSpec((1, H), lambda i: (0, 0)),
    ],
    out_specs=pl.BlockSpec((BLK, H), lambda i: (i, 0)),
    out_shape=jax.ShapeDtypeStruct((N, H), jnp.float32),
)


NP = N // 2
DP = 2 * H
BLKP = 1000


def _l2mlp_body(p_ref, x_ref, wr_ref, wo_ref, b_ref, w1_ref, b1_ref,
                w2_ref, b2_ref, out_ref, hr_ref):
    pp = p_ref[...]
    agg = pp[0] + pp[1]
    h = jnp.maximum(
        jnp.dot(agg, wr_ref[...], preferred_element_type=jnp.float32)
        + b_ref[...]
        + jnp.dot(x_ref[...], wo_ref[...], preferred_element_type=jnp.float32),
        0.0)
    hr_ref[...] = h
    hid = jnp.maximum(
        jnp.dot(h, w1_ref[...], preferred_element_type=jnp.float32)
        + b1_ref[...], 0.0)
    z = jnp.dot(hid, w2_ref[...], preferred_element_type=jnp.float32) + b2_ref[...]
    out_ref[...] = jax.nn.sigmoid(z)


_l2mlp = pl.pallas_call(
    _l2mlp_body,
    grid=(NP // BLKP,),
    in_specs=[
        pl.BlockSpec((NC, BLKP, DP), lambda i: (0, i, 0)),
        pl.BlockSpec((BLKP, DP), lambda i: (i, 0)),
        pl.BlockSpec((DP, DP), lambda i: (0, 0)),
        pl.BlockSpec((DP, DP), lambda i: (0, 0)),
        pl.BlockSpec((1, DP), lambda i: (0, 0)),
        pl.BlockSpec((DP, 3 * H), lambda i: (0, 0)),
        pl.BlockSpec((1, 3 * H), lambda i: (0, 0)),
        pl.BlockSpec((3 * H, OUT), lambda i: (0, 0)),
        pl.BlockSpec((1, OUT), lambda i: (0, 0)),
    ],
    out_specs=[
        pl.BlockSpec((BLKP, OUT), lambda i: (i, 0)),
        pl.BlockSpec((BLKP, DP), lambda i: (i, 0)),
    ],
    out_shape=[
        jax.ShapeDtypeStruct((NP, OUT), jnp.float32),
        jax.ShapeDtypeStruct((NP, DP), jnp.float32),
    ],
)


def kernel(x, edge_index, batch, W_rel0, b_rel0, W_root0, W_rel1, b_rel1,
           W_root1, mlp_W1, mlp_b1, mlp_W2, mlp_b2):
    src = edge_index[0]
    dst = edge_index[1]

    parts0 = _segsum(x, src, dst)
    h1 = _layer(parts0, x, W_rel0, W_root0, b_rel0.reshape(1, H))
    parts1 = _segsum(h1, src, dst)

    z = jnp.zeros((H, H), jnp.float32)
    wr1p = jnp.concatenate(
        [jnp.concatenate([W_rel1, z], 1), jnp.concatenate([z, W_rel1], 1)], 0)
    wo1p = jnp.concatenate(
        [jnp.concatenate([W_root1, z], 1), jnp.concatenate([z, W_root1], 1)], 0)
    b1p = jnp.concatenate([b_rel1, b_rel1]).reshape(1, DP)
    out, hr = _l2mlp(parts1.reshape(NC, NP, DP), h1.reshape(NP, DP),
                     wr1p, wo1p, b1p, mlp_W1, mlp_b1.reshape(1, 3 * H),
                     mlp_W2, mlp_b2.reshape(1, OUT))
    return (out, hr, hr[:, :H], hr[:, H:])


# speedup vs baseline: 1.0006x; 1.0006x over previous
import jax
import jax.numpy as jnp
from jax import lax
from jax.experimental import pallas as pl
from jax.experimental.pallas import tpu as pltpu
from jax.experimental.pallas import tpu_sc as plsc

N = 10000
E = 320000
D = 128
H = 128
OUT = 16

NC = 2
NS = 16
EPW = E // (NC * NS)
CH = 80
NCHUNK = EPW // CH
RPW = 624
EXTRA_ROW0 = NS * RPW
RCH = 208
NRCH = RPW // RCH


def _segsum_body(m_hbm, src_hbm, dst_hbm, parts_hbm,
                 acc, src0, dst0, src1, dst1, src2, dst2, src3, dst3,
                 rows0, rows1, zbuf,
                 semi0, semi1, semi2, semi3, semg0, semg1, sems0, sems1):
    c = lax.axis_index("c")
    s = lax.axis_index("s")
    row0 = s * RPW
    erow = EXTRA_ROW0 + s * 8
    SRC = (src0, src1, src2, src3)
    DST = (dst0, dst1, dst2, dst3)
    ROWS = (rows0, rows1)
    SEMI = (semi0, semi1, semi2, semi3)
    SEMG = (semg0, semg1)
    SEMS = (sems0, sems1)

    def zrow(r, carry):
        for u in range(8):
            zbuf[r, pl.ds(u * 16, 16)] = jnp.zeros((16,), jnp.float32)
        return carry
    lax.fori_loop(0, RCH, zrow, 0)

    def zacc(j, carry):
        pltpu.sync_copy(zbuf, acc.at[pl.ds(row0 + j * RCH, RCH)])
        return carry
    lax.fori_loop(0, NRCH, zacc, 0)

    @pl.when(s < 2)
    def _():
        pltpu.sync_copy(zbuf.at[pl.ds(0, 8)], acc.at[pl.ds(erow, 8)])
    plsc.subcore_barrier()

    base0 = (c * NS + s) * EPW

    def issue_idx(j, q):
        b = base0 + jnp.minimum(j, NCHUNK - 1) * CH
        pltpu.async_copy(src_hbm.at[pl.ds(b, CH)], SRC[q], SEMI[q])
        pltpu.async_copy(dst_hbm.at[pl.ds(b, CH)], DST[q], SEMI[q])

    def wait_idx(q):
        pltpu.make_async_copy(src_hbm.at[pl.ds(0, CH)], SRC[q], SEMI[q]).wait()
        pltpu.make_async_copy(dst_hbm.at[pl.ds(0, CH)], DST[q], SEMI[q]).wait()

    def issue_gather(p, q):
        pltpu.async_copy(m_hbm.at[SRC[q]], ROWS[p], SEMG[p])

    def wait_gather(p, q):
        pltpu.make_async_copy(m_hbm.at[SRC[q]], ROWS[p], SEMG[p]).wait()

    def issue_scatter(p, q):
        pltpu.async_copy(ROWS[p], acc.at[DST[q]], SEMS[p], add=True)

    def wait_scatter(p, q):
        pltpu.make_async_copy(ROWS[p], acc.at[DST[q]], SEMS[p]).wait()

    def hstep(j, q, p):
        wait_idx((q + 1) % 4)
        wait_scatter(1 - p, (q + 3) % 4)
        issue_gather(1 - p, (q + 1) % 4)
        issue_idx(j + 3, (q + 3) % 4)
        wait_gather(p, q)
        issue_scatter(p, q)

    issue_idx(0, 0)
    issue_idx(1, 1)
    issue_idx(2, 2)
    wait_idx(0)
    issue_gather(0, 0)
    wait_idx(1)
    issue_gather(1, 1)
    issue_idx(3, 3)
    wait_gather(0, 0)
    issue_scatter(0, 0)

    def step4(t, carry):
        j = 4 * t + 1
        hstep(j, 1, 1)
        hstep(j + 1, 2, 0)
        hstep(j + 2, 3, 1)
        hstep(j + 3, 0, 0)
        return carry
    lax.fori_loop(0, (NCHUNK - 1) // 4, step4, 0)

    wait_idx(2)
    wait_idx(3)
    wait_gather(1, 1)
    wait_scatter(0, 0)
    plsc.subcore_barrier()

    def wb(j, carry):
        r0 = row0 + j * RCH
        pltpu.sync_copy(acc.at[pl.ds(r0, RCH)], zbuf)
        pltpu.sync_copy(zbuf, parts_hbm.at[c, pl.ds(r0, RCH)])
        return carry
    lax.fori_loop(0, NRCH, wb, 0)

    @pl.when(s < 2)
    def _():
        pltpu.sync_copy(acc.at[pl.ds(erow, 8)], parts_hbm.at[c, pl.ds(erow, 8)])


_segsum = pl.kernel(
    _segsum_body,
    out_type=jax.ShapeDtypeStruct((NC, N, H), jnp.float32),
    mesh=plsc.VectorSubcoreMesh(core_axis_name="c", subcore_axis_name="s",
                                num_cores=NC, num_subcores=NS),
    scratch_types=[
        pltpu.VMEM_SHARED((N, H), jnp.float32),
        pltpu.VMEM((CH,), jnp.int32),
        pltpu.VMEM((CH,), jnp.int32),
        pltpu.VMEM((CH,), jnp.int32),
        pltpu.VMEM((CH,), jnp.int32),
        pltpu.VMEM((CH,), jnp.int32),
        pltpu.VMEM((CH,), jnp.int32),
        pltpu.VMEM((CH,), jnp.int32),
        pltpu.VMEM((CH,), jnp.int32),
        pltpu.VMEM((CH, H), jnp.float32),
        pltpu.VMEM((CH, H), jnp.float32),
        pltpu.VMEM((RCH, H), jnp.float32),
        pltpu.SemaphoreType.DMA,
        pltpu.SemaphoreType.DMA,
        pltpu.SemaphoreType.DMA,
        pltpu.SemaphoreType.DMA,
        pltpu.SemaphoreType.DMA,
        pltpu.SemaphoreType.DMA,
        pltpu.SemaphoreType.DMA,
        pltpu.SemaphoreType.DMA,
    ],
)


BLK = 2000


def _layer_body(p_ref, x_ref, wr_ref, wo_ref, b_ref, h_ref):
    pp = p_ref[...]
    agg = pp[0] + pp[1]
    h_ref[...] = jnp.maximum(
        jnp.dot(agg, wr_ref[...], preferred_element_type=jnp.float32)
        + b_ref[...]
        + jnp.dot(x_ref[...], wo_ref[...], preferred_element_type=jnp.float32),
        0.0)


_layer = pl.pallas_call(
    _layer_body,
    grid=(N // BLK,),
    in_specs=[
        pl.BlockSpec((NC, BLK, H), lambda i: (0, i, 0)),
        pl.BlockSpec((BLK, H), lambda i: (i, 0)),
        pl.BlockSpec((H, H), lambda i: (0, 0)),
        pl.BlockSpec((H, H), lambda i: (0, 0)),
        pl.BlockSpec((1, H), lambda i: (0, 0)),
    ],
    out_specs=pl.BlockSpec((BLK, H), lambda i: (i, 0)),
    out_shape=jax.ShapeDtypeStruct((N, H), jnp.float32),
)


NP = N // 2
DP = 2 * H
BLKP = 1000


def _l2mlp_body(p_ref, x_ref, wr_ref, wo_ref, b_ref, w1_ref, b1_ref,
                w2_ref, b2_ref, out_ref, hr_ref):
    pp = p_ref[...]
    agg = pp[0] + pp[1]
    h = jnp.maximum(
        jnp.dot(agg, wr_ref[...], preferred_element_type=jnp.float32)
        + b_ref[...]
        + jnp.dot(x_ref[...], wo_ref[...], preferred_element_type=jnp.float32),
        0.0)
    hr_ref[...] = h
    hid = jnp.maximum(
        jnp.dot(h, w1_ref[...], preferred_element_type=jnp.float32)
        + b1_ref[...], 0.0)
    z = jnp.dot(hid, w2_ref[...], preferred_element_type=jnp.float32) + b2_ref[...]
    out_ref[...] = jax.nn.sigmoid(z)


_l2mlp = pl.pallas_call(
    _l2mlp_body,
    grid=(NP // BLKP,),
    in_specs=[
        pl.BlockSpec((NC, BLKP, DP), lambda i: (0, i, 0)),
        pl.BlockSpec((BLKP, DP), lambda i: (i, 0)),
        pl.BlockSpec((DP, DP), lambda i: (0, 0)),
        pl.BlockSpec((DP, DP), lambda i: (0, 0)),
        pl.BlockSpec((1, DP), lambda i: (0, 0)),
        pl.BlockSpec((DP, 3 * H), lambda i: (0, 0)),
        pl.BlockSpec((1, 3 * H), lambda i: (0, 0)),
        pl.BlockSpec((3 * H, OUT), lambda i: (0, 0)),
        pl.BlockSpec((1, OUT), lambda i: (0, 0)),
    ],
    out_specs=[
        pl.BlockSpec((BLKP, OUT), lambda i: (i, 0)),
        pl.BlockSpec((BLKP, DP), lambda i: (i, 0)),
    ],
    out_shape=[
        jax.ShapeDtypeStruct((NP, OUT), jnp.float32),
        jax.ShapeDtypeStruct((NP, DP), jnp.float32),
    ],
)


def kernel(x, edge_index, batch, W_rel0, b_rel0, W_root0, W_rel1, b_rel1,
           W_root1, mlp_W1, mlp_b1, mlp_W2, mlp_b2):
    src = edge_index[0]
    dst = edge_index[1]

    parts0 = _segsum(x, src, dst)
    h1 = _layer(parts0, x, W_rel0, W_root0, b_rel0.reshape(1, H))
    parts1 = _segsum(h1, src, dst)

    z = jnp.zeros((H, H), jnp.float32)
    wr1p = jnp.concatenate(
        [jnp.concatenate([W_rel1, z], 1), jnp.concatenate([z, W_rel1], 1)], 0)
    wo1p = jnp.concatenate(
        [jnp.concatenate([W_root1, z], 1), jnp.concatenate([z, W_root1], 1)], 0)
    b1p = jnp.concatenate([b_rel1, b_rel1]).reshape(1, DP)
    out, hr = _l2mlp(parts1.reshape(NC, NP, DP), h1.reshape(NP, DP),
                     wr1p, wo1p, b1p, mlp_W1, mlp_b1.reshape(1, 3 * H),
                     mlp_W2, mlp_b2.reshape(1, OUT))
    return (out, hr, hr[:, :H], hr[:, H:])


# speedup vs baseline: 1.0100x; 1.0093x over previous
import jax
import jax.numpy as jnp
from jax import lax
from jax.experimental import pallas as pl
from jax.experimental.pallas import tpu as pltpu
from jax.experimental.pallas import tpu_sc as plsc

N = 10000
E = 320000
D = 128
H = 128
OUT = 16

NC = 2
NS = 16
EPW = E // (NC * NS)
CH = 80
NCHUNK = EPW // CH
RPW = 624
EXTRA_ROW0 = NS * RPW
RCH = 208
NRCH = RPW // RCH


def _segsum_body(m_hbm, src_hbm, dst_hbm, parts_hbm,
                 acc, src0, dst0, src1, dst1, src2, dst2, src3, dst3,
                 rows0, rows1, zbuf,
                 semi0, semi1, semi2, semi3, semg0, semg1, sems0, sems1):
    c = lax.axis_index("c")
    s = lax.axis_index("s")
    row0 = s * RPW
    erow = EXTRA_ROW0 + s * 8
    SRC = (src0, src1, src2, src3)
    DST = (dst0, dst1, dst2, dst3)
    ROWS = (rows0, rows1)
    SEMI = (semi0, semi1, semi2, semi3)
    SEMG = (semg0, semg1)
    SEMS = (sems0, sems1)

    def zrow(r, carry):
        for u in range(8):
            zbuf[r, pl.ds(u * 16, 16)] = jnp.zeros((16,), jnp.float32)
        return carry
    lax.fori_loop(0, RCH, zrow, 0)

    def zacc(j, carry):
        pltpu.sync_copy(zbuf, acc.at[pl.ds(row0 + j * RCH, RCH)])
        return carry
    lax.fori_loop(0, NRCH, zacc, 0)

    @pl.when(s < 2)
    def _():
        pltpu.sync_copy(zbuf.at[pl.ds(0, 8)], acc.at[pl.ds(erow, 8)])
    plsc.subcore_barrier()

    base0 = (c * NS + s) * EPW

    def issue_idx(j, q):
        b = base0 + jnp.minimum(j, NCHUNK - 1) * CH
        pltpu.async_copy(src_hbm.at[pl.ds(b, CH)], SRC[q], SEMI[q])
        pltpu.async_copy(dst_hbm.at[pl.ds(b, CH)], DST[q], SEMI[q])

    def wait_idx(q):
        pltpu.make_async_copy(src_hbm.at[pl.ds(0, CH)], SRC[q], SEMI[q]).wait()
        pltpu.make_async_copy(dst_hbm.at[pl.ds(0, CH)], DST[q], SEMI[q]).wait()

    def issue_gather(p, q):
        pltpu.async_copy(m_hbm.at[SRC[q]], ROWS[p], SEMG[p])

    def wait_gather(p, q):
        pltpu.make_async_copy(m_hbm.at[SRC[q]], ROWS[p], SEMG[p]).wait()

    def issue_scatter(p, q):
        pltpu.async_copy(ROWS[p], acc.at[DST[q]], SEMS[p], add=True)

    def wait_scatter(p, q):
        pltpu.make_async_copy(ROWS[p], acc.at[DST[q]], SEMS[p]).wait()

    def hstep(j, q, p):
        wait_idx((q + 1) % 4)
        wait_scatter(1 - p, (q + 3) % 4)
        issue_gather(1 - p, (q + 1) % 4)
        issue_idx(j + 3, (q + 3) % 4)
        wait_gather(p, q)
        issue_scatter(p, q)

    issue_idx(0, 0)
    issue_idx(1, 1)
    issue_idx(2, 2)
    wait_idx(0)
    issue_gather(0, 0)
    wait_idx(1)
    issue_gather(1, 1)
    issue_idx(3, 3)
    wait_gather(0, 0)
    issue_scatter(0, 0)

    def step4(t, carry):
        j = 4 * t + 1
        hstep(j, 1, 1)
        hstep(j + 1, 2, 0)
        hstep(j + 2, 3, 1)
        hstep(j + 3, 0, 0)
        return carry
    lax.fori_loop(0, (NCHUNK - 1) // 4, step4, 0)

    wait_idx(2)
    wait_idx(3)
    wait_gather(1, 1)
    wait_scatter(0, 0)
    plsc.subcore_barrier()

    pltpu.sync_copy(acc.at[pl.ds(row0, RPW)], parts_hbm.at[c, pl.ds(row0, RPW)])

    @pl.when(s < 2)
    def _():
        pltpu.sync_copy(acc.at[pl.ds(erow, 8)], parts_hbm.at[c, pl.ds(erow, 8)])


_segsum = pl.kernel(
    _segsum_body,
    out_type=jax.ShapeDtypeStruct((NC, N, H), jnp.float32),
    mesh=plsc.VectorSubcoreMesh(core_axis_name="c", subcore_axis_name="s",
                                num_cores=NC, num_subcores=NS),
    scratch_types=[
        pltpu.VMEM_SHARED((N, H), jnp.float32),
        pltpu.VMEM((CH,), jnp.int32),
        pltpu.VMEM((CH,), jnp.int32),
        pltpu.VMEM((CH,), jnp.int32),
        pltpu.VMEM((CH,), jnp.int32),
        pltpu.VMEM((CH,), jnp.int32),
        pltpu.VMEM((CH,), jnp.int32),
        pltpu.VMEM((CH,), jnp.int32),
        pltpu.VMEM((CH,), jnp.int32),
        pltpu.VMEM((CH, H), jnp.float32),
        pltpu.VMEM((CH, H), jnp.float32),
        pltpu.VMEM((RCH, H), jnp.float32),
        pltpu.SemaphoreType.DMA,
        pltpu.SemaphoreType.DMA,
        pltpu.SemaphoreType.DMA,
        pltpu.SemaphoreType.DMA,
        pltpu.SemaphoreType.DMA,
        pltpu.SemaphoreType.DMA,
        pltpu.SemaphoreType.DMA,
        pltpu.SemaphoreType.DMA,
    ],
)


BLK = 2000


def _layer_body(p_ref, x_ref, wr_ref, wo_ref, b_ref, h_ref):
    pp = p_ref[...]
    agg = pp[0] + pp[1]
    h_ref[...] = jnp.maximum(
        jnp.dot(agg, wr_ref[...], preferred_element_type=jnp.float32)
        + b_ref[...]
        + jnp.dot(x_ref[...], wo_ref[...], preferred_element_type=jnp.float32),
        0.0)


_layer = pl.pallas_call(
    _layer_body,
    grid=(N // BLK,),
    in_specs=[
        pl.BlockSpec((NC, BLK, H), lambda i: (0, i, 0)),
        pl.BlockSpec((BLK, H), lambda i: (i, 0)),
        pl.BlockSpec((H, H), lambda i: (0, 0)),
        pl.BlockSpec((H, H), lambda i: (0, 0)),
        pl.BlockSpec((1, H), lambda i: (0, 0)),
    ],
    out_specs=pl.BlockSpec((BLK, H), lambda i: (i, 0)),
    out_shape=jax.ShapeDtypeStruct((N, H), jnp.float32),
)


NP = N // 2
DP = 2 * H
BLKP = 1000


def _mlp_body(hr_ref, w1_ref, b1_ref, w2_ref, b2_ref, out_ref):
    hid = jnp.maximum(
        jnp.dot(hr_ref[...], w1_ref[...], preferred_element_type=jnp.float32)
        + b1_ref[...], 0.0)
    z = jnp.dot(hid, w2_ref[...], preferred_element_type=jnp.float32) + b2_ref[...]
    out_ref[...] = jax.nn.sigmoid(z)


_mlp = pl.pallas_call(
    _mlp_body,
    grid=(NP // BLKP,),
    in_specs=[
        pl.BlockSpec((BLKP, DP), lambda i: (i, 0)),
        pl.BlockSpec((DP, 3 * H), lambda i: (0, 0)),
        pl.BlockSpec((1, 3 * H), lambda i: (0, 0)),
        pl.BlockSpec((3 * H, OUT), lambda i: (0, 0)),
        pl.BlockSpec((1, OUT), lambda i: (0, 0)),
    ],
    out_specs=pl.BlockSpec((BLKP, OUT), lambda i: (i, 0)),
    out_shape=jax.ShapeDtypeStruct((NP, OUT), jnp.float32),
)


def kernel(x, edge_index, batch, W_rel0, b_rel0, W_root0, W_rel1, b_rel1,
           W_root1, mlp_W1, mlp_b1, mlp_W2, mlp_b2):
    src = edge_index[0]
    dst = edge_index[1]

    parts0 = _segsum(x, src, dst)
    h1 = _layer(parts0, x, W_rel0, W_root0, b_rel0.reshape(1, H))
    parts1 = _segsum(h1, src, dst)
    h2 = _layer(parts1, h1, W_rel1, W_root1, b_rel1.reshape(1, H))

    hr = h2.reshape(NP, DP)
    out = _mlp(hr, mlp_W1, mlp_b1.reshape(1, 3 * H), mlp_W2,
               mlp_b2.reshape(1, OUT))
    return (out, hr, hr[:, :H], hr[:, H:])


# speedup vs baseline: 1.0874x; 1.0766x over previous
import jax
import jax.numpy as jnp
from jax import lax
from jax.experimental import pallas as pl
from jax.experimental.pallas import tpu as pltpu
from jax.experimental.pallas import tpu_sc as plsc

N = 10000
E = 320000
D = 128
H = 128
OUT = 16

NC = 2
NS = 16
EPW = E // (NC * NS)
CH = 128
NF = EPW // CH
TAIL = EPW - NF * CH
RPW = 624
EXTRA_ROW0 = NS * RPW
RCH = 104
NRCH = RPW // RCH


def _segsum_body(m_hbm, src_hbm, dst_hbm, parts_hbm,
                 acc, src0, dst0, src1, dst1, src2, dst2, src3, dst3,
                 srct, dstt, rows0, rows1, zbuf,
                 semi0, semi1, semi2, semi3, semg0, semg1, sems0, sems1):
    c = lax.axis_index("c")
    s = lax.axis_index("s")
    row0 = s * RPW
    erow = EXTRA_ROW0 + s * 8
    SRC = (src0, src1, src2, src3)
    DST = (dst0, dst1, dst2, dst3)
    ROWS = (rows0, rows1)
    SEMI = (semi0, semi1, semi2, semi3)
    SEMG = (semg0, semg1)
    SEMS = (sems0, sems1)

    def zrow(r, carry):
        for u in range(8):
            zbuf[r, pl.ds(u * 16, 16)] = jnp.zeros((16,), jnp.float32)
        return carry
    lax.fori_loop(0, RCH, zrow, 0)

    def zacc(j, carry):
        pltpu.sync_copy(zbuf, acc.at[pl.ds(row0 + j * RCH, RCH)])
        return carry
    lax.fori_loop(0, NRCH, zacc, 0)

    @pl.when(s < 2)
    def _():
        pltpu.sync_copy(zbuf.at[pl.ds(0, 8)], acc.at[pl.ds(erow, 8)])
    plsc.subcore_barrier()

    base0 = (c * NS + s) * EPW

    def issue_idx(j, q):
        b = base0 + jnp.minimum(j, NF - 1) * CH
        pltpu.async_copy(src_hbm.at[pl.ds(b, CH)], SRC[q], SEMI[q])
        pltpu.async_copy(dst_hbm.at[pl.ds(b, CH)], DST[q], SEMI[q])

    def wait_idx(q):
        pltpu.make_async_copy(src_hbm.at[pl.ds(0, CH)], SRC[q], SEMI[q]).wait()
        pltpu.make_async_copy(dst_hbm.at[pl.ds(0, CH)], DST[q], SEMI[q]).wait()

    def issue_gather(p, q):
        pltpu.async_copy(m_hbm.at[SRC[q]], ROWS[p], SEMG[p])

    def wait_gather(p, q):
        pltpu.make_async_copy(m_hbm.at[SRC[q]], ROWS[p], SEMG[p]).wait()

    def issue_scatter(p, q):
        pltpu.async_copy(ROWS[p], acc.at[DST[q]], SEMS[p], add=True)

    def wait_scatter(p, q):
        pltpu.make_async_copy(ROWS[p], acc.at[DST[q]], SEMS[p]).wait()

    def hstep(j, q, p):
        wait_idx((q + 1) % 4)
        wait_scatter(1 - p, (q + 3) % 4)
        issue_gather(1 - p, (q + 1) % 4)
        issue_idx(j + 3, (q + 3) % 4)
        wait_gather(p, q)
        issue_scatter(p, q)

    issue_idx(0, 0)
    issue_idx(1, 1)
    issue_idx(2, 2)
    wait_idx(0)
    issue_gather(0, 0)
    wait_idx(1)
    issue_gather(1, 1)
    issue_idx(3, 3)
    wait_gather(0, 0)
    issue_scatter(0, 0)
    hstep(1, 1, 1)

    def step4(t, carry):
        j = 4 * t + 2
        hstep(j, 2, 0)
        hstep(j + 1, 3, 1)
        hstep(j + 2, 0, 0)
        hstep(j + 3, 1, 1)
        return carry
    lax.fori_loop(0, (NF - 2) // 4, step4, 0)

    wait_idx(3)
    wait_idx(0)
    wait_gather(0, 2)
    wait_scatter(1, 1)

    bt = base0 + NF * CH
    pltpu.sync_copy(src_hbm.at[pl.ds(bt, TAIL)], srct)
    pltpu.sync_copy(dst_hbm.at[pl.ds(bt, TAIL)], dstt)
    pltpu.async_copy(m_hbm.at[srct], rows0.at[pl.ds(0, TAIL)], semg0).wait()
    pltpu.sync_copy(rows0.at[pl.ds(0, TAIL)], acc.at[dstt], add=True)
    plsc.subcore_barrier()

    pltpu.sync_copy(acc.at[pl.ds(row0, RPW)], parts_hbm.at[c, pl.ds(row0, RPW)])

    @pl.when(s < 2)
    def _():
        pltpu.sync_copy(acc.at[pl.ds(erow, 8)], parts_hbm.at[c, pl.ds(erow, 8)])


_segsum = pl.kernel(
    _segsum_body,
    out_type=jax.ShapeDtypeStruct((NC, N, H), jnp.float32),
    mesh=plsc.VectorSubcoreMesh(core_axis_name="c", subcore_axis_name="s",
                                num_cores=NC, num_subcores=NS),
    scratch_types=[
        pltpu.VMEM_SHARED((N, H), jnp.float32),
        pltpu.VMEM((CH,), jnp.int32),
        pltpu.VMEM((CH,), jnp.int32),
        pltpu.VMEM((CH,), jnp.int32),
        pltpu.VMEM((CH,), jnp.int32),
        pltpu.VMEM((CH,), jnp.int32),
        pltpu.VMEM((CH,), jnp.int32),
        pltpu.VMEM((CH,), jnp.int32),
        pltpu.VMEM((CH,), jnp.int32),
        pltpu.VMEM((TAIL,), jnp.int32),
        pltpu.VMEM((TAIL,), jnp.int32),
        pltpu.VMEM((CH, H), jnp.float32),
        pltpu.VMEM((CH, H), jnp.float32),
        pltpu.VMEM((RCH, H), jnp.float32),
        pltpu.SemaphoreType.DMA,
        pltpu.SemaphoreType.DMA,
        pltpu.SemaphoreType.DMA,
        pltpu.SemaphoreType.DMA,
        pltpu.SemaphoreType.DMA,
        pltpu.SemaphoreType.DMA,
        pltpu.SemaphoreType.DMA,
        pltpu.SemaphoreType.DMA,
    ],
)


BLK = 2000


def _layer_body(p_ref, x_ref, wr_ref, wo_ref, b_ref, h_ref):
    pp = p_ref[...]
    agg = pp[0] + pp[1]
    h_ref[...] = jnp.maximum(
        jnp.dot(agg, wr_ref[...], preferred_element_type=jnp.float32)
        + b_ref[...]
        + jnp.dot(x_ref[...], wo_ref[...], preferred_element_type=jnp.float32),
        0.0)


_layer = pl.pallas_call(
    _layer_body,
    grid=(N // BLK,),
    in_specs=[
        pl.BlockSpec((NC, BLK, H), lambda i: (0, i, 0)),
        pl.BlockSpec((BLK, H), lambda i: (i, 0)),
        pl.BlockSpec((H, H), lambda i: (0, 0)),
        pl.BlockSpec((H, H), lambda i: (0, 0)),
        pl.BlockSpec((1, H), lambda i: (0, 0)),
    ],
    out_specs=pl.BlockSpec((BLK, H), lambda i: (i, 0)),
    out_shape=jax.ShapeDtypeStruct((N, H), jnp.float32),
)


NP = N // 2
DP = 2 * H
BLKP = 1000


def _mlp_body(hr_ref, w1_ref, b1_ref, w2_ref, b2_ref, out_ref):
    hid = jnp.maximum(
        jnp.dot(hr_ref[...], w1_ref[...], preferred_element_type=jnp.float32)
        + b1_ref[...], 0.0)
    z = jnp.dot(hid, w2_ref[...], preferred_element_type=jnp.float32) + b2_ref[...]
    out_ref[...] = jax.nn.sigmoid(z)


_mlp = pl.pallas_call(
    _mlp_body,
    grid=(NP // BLKP,),
    in_specs=[
        pl.BlockSpec((BLKP, DP), lambda i: (i, 0)),
        pl.BlockSpec((DP, 3 * H), lambda i: (0, 0)),
        pl.BlockSpec((1, 3 * H), lambda i: (0, 0)),
        pl.BlockSpec((3 * H, OUT), lambda i: (0, 0)),
        pl.BlockSpec((1, OUT), lambda i: (0, 0)),
    ],
    out_specs=pl.BlockSpec((BLKP, OUT), lambda i: (i, 0)),
    out_shape=jax.ShapeDtypeStruct((NP, OUT), jnp.float32),
)


def kernel(x, edge_index, batch, W_rel0, b_rel0, W_root0, W_rel1, b_rel1,
           W_root1, mlp_W1, mlp_b1, mlp_W2, mlp_b2):
    src = edge_index[0]
    dst = edge_index[1]

    parts0 = _segsum(x, src, dst)
    h1 = _layer(parts0, x, W_rel0, W_root0, b_rel0.reshape(1, H))
    parts1 = _segsum(h1, src, dst)
    h2 = _layer(parts1, h1, W_rel1, W_root1, b_rel1.reshape(1, H))

    hr = h2.reshape(NP, DP)
    out = _mlp(hr, mlp_W1, mlp_b1.reshape(1, 3 * H), mlp_W2,
               mlp_b2.reshape(1, OUT))
    return (out, hr, hr[:, :H], hr[:, H:])


# speedup vs baseline: 1.1022x; 1.0136x over previous
import jax
import jax.numpy as jnp
from jax import lax
from jax.experimental import pallas as pl
from jax.experimental.pallas import tpu as pltpu
from jax.experimental.pallas import tpu_sc as plsc

N = 10000
E = 320000
D = 128
H = 128
OUT = 16

NC = 2
NS = 16
EPW = E // (NC * NS)
CH = 128
NF = EPW // CH
TAIL = EPW - NF * CH
RPW = 624
EXTRA_ROW0 = NS * RPW
RCH = 104
NRCH = RPW // RCH


def _segsum_body(m_hbm, src_hbm, dst_hbm, parts_hbm,
                 acc, src0, dst0, src1, dst1, src2, dst2, src3, dst3,
                 srct, dstt, rows0, rows1, zbuf,
                 semi0, semi1, semi2, semi3, semg0, semg1, sems0, sems1):
    c = lax.axis_index("c")
    s = lax.axis_index("s")
    row0 = s * RPW
    erow = EXTRA_ROW0 + s * 8
    SRC = (src0, src1, src2, src3)
    DST = (dst0, dst1, dst2, dst3)
    ROWS = (rows0, rows1)
    SEMI = (semi0, semi1, semi2, semi3)
    SEMG = (semg0, semg1)
    SEMS = (sems0, sems1)

    base0 = (c * NS + s) * EPW

    def issue_idx(j, q):
        b = base0 + jnp.minimum(j, NF - 1) * CH
        pltpu.async_copy(src_hbm.at[pl.ds(b, CH)], SRC[q], SEMI[q])
        pltpu.async_copy(dst_hbm.at[pl.ds(b, CH)], DST[q], SEMI[q])

    def wait_idx(q):
        pltpu.make_async_copy(src_hbm.at[pl.ds(0, CH)], SRC[q], SEMI[q]).wait()
        pltpu.make_async_copy(dst_hbm.at[pl.ds(0, CH)], DST[q], SEMI[q]).wait()

    def issue_gather(p, q):
        pltpu.async_copy(m_hbm.at[SRC[q]], ROWS[p], SEMG[p])

    def wait_gather(p, q):
        pltpu.make_async_copy(m_hbm.at[SRC[q]], ROWS[p], SEMG[p]).wait()

    def issue_scatter(p, q):
        pltpu.async_copy(ROWS[p], acc.at[DST[q]], SEMS[p], add=True)

    def wait_scatter(p, q):
        pltpu.make_async_copy(ROWS[p], acc.at[DST[q]], SEMS[p]).wait()

    def hstep(j, q, p):
        wait_idx((q + 1) % 4)
        wait_scatter(1 - p, (q + 3) % 4)
        issue_gather(1 - p, (q + 1) % 4)
        issue_idx(j + 3, (q + 3) % 4)
        wait_gather(p, q)
        issue_scatter(p, q)

    issue_idx(0, 0)
    issue_idx(1, 1)
    issue_idx(2, 2)
    wait_idx(0)
    issue_gather(0, 0)
    wait_idx(1)
    issue_gather(1, 1)
    issue_idx(3, 3)

    def zrow(r, carry):
        for u in range(8):
            zbuf[r, pl.ds(u * 16, 16)] = jnp.zeros((16,), jnp.float32)
        return carry
    lax.fori_loop(0, RCH, zrow, 0)

    def zacc(j, carry):
        pltpu.sync_copy(zbuf, acc.at[pl.ds(row0 + j * RCH, RCH)])
        return carry
    lax.fori_loop(0, NRCH, zacc, 0)

    @pl.when(s < 2)
    def _():
        pltpu.sync_copy(zbuf.at[pl.ds(0, 8)], acc.at[pl.ds(erow, 8)])
    plsc.subcore_barrier()

    wait_gather(0, 0)
    issue_scatter(0, 0)
    hstep(1, 1, 1)

    def step4(t, carry):
        j = 4 * t + 2
        hstep(j, 2, 0)
        hstep(j + 1, 3, 1)
        hstep(j + 2, 0, 0)
        hstep(j + 3, 1, 1)
        return carry
    lax.fori_loop(0, (NF - 2) // 4, step4, 0)

    wait_idx(3)
    wait_idx(0)
    wait_gather(0, 2)
    wait_scatter(1, 1)

    bt = base0 + NF * CH
    pltpu.sync_copy(src_hbm.at[pl.ds(bt, TAIL)], srct)
    pltpu.sync_copy(dst_hbm.at[pl.ds(bt, TAIL)], dstt)
    pltpu.async_copy(m_hbm.at[srct], rows0.at[pl.ds(0, TAIL)], semg0).wait()
    pltpu.sync_copy(rows0.at[pl.ds(0, TAIL)], acc.at[dstt], add=True)
    plsc.subcore_barrier()

    pltpu.sync_copy(acc.at[pl.ds(row0, RPW)], parts_hbm.at[c, pl.ds(row0, RPW)])

    @pl.when(s < 2)
    def _():
        pltpu.sync_copy(acc.at[pl.ds(erow, 8)], parts_hbm.at[c, pl.ds(erow, 8)])


_segsum = pl.kernel(
    _segsum_body,
    out_type=jax.ShapeDtypeStruct((NC, N, H), jnp.float32),
    mesh=plsc.VectorSubcoreMesh(core_axis_name="c", subcore_axis_name="s",
                                num_cores=NC, num_subcores=NS),
    scratch_types=[
        pltpu.VMEM_SHARED((N, H), jnp.float32),
        pltpu.VMEM((CH,), jnp.int32),
        pltpu.VMEM((CH,), jnp.int32),
        pltpu.VMEM((CH,), jnp.int32),
        pltpu.VMEM((CH,), jnp.int32),
        pltpu.VMEM((CH,), jnp.int32),
        pltpu.VMEM((CH,), jnp.int32),
        pltpu.VMEM((CH,), jnp.int32),
        pltpu.VMEM((CH,), jnp.int32),
        pltpu.VMEM((TAIL,), jnp.int32),
        pltpu.VMEM((TAIL,), jnp.int32),
        pltpu.VMEM((CH, H), jnp.float32),
        pltpu.VMEM((CH, H), jnp.float32),
        pltpu.VMEM((RCH, H), jnp.float32),
        pltpu.SemaphoreType.DMA,
        pltpu.SemaphoreType.DMA,
        pltpu.SemaphoreType.DMA,
        pltpu.SemaphoreType.DMA,
        pltpu.SemaphoreType.DMA,
        pltpu.SemaphoreType.DMA,
        pltpu.SemaphoreType.DMA,
        pltpu.SemaphoreType.DMA,
    ],
)


BLK = 2000


def _layer_body(p_ref, x_ref, wr_ref, wo_ref, b_ref, h_ref):
    pp = p_ref[...]
    agg = pp[0] + pp[1]
    h_ref[...] = jnp.maximum(
        jnp.dot(agg, wr_ref[...], preferred_element_type=jnp.float32)
        + b_ref[...]
        + jnp.dot(x_ref[...], wo_ref[...], preferred_element_type=jnp.float32),
        0.0)


_layer = pl.pallas_call(
    _layer_body,
    grid=(N // BLK,),
    in_specs=[
        pl.BlockSpec((NC, BLK, H), lambda i: (0, i, 0)),
        pl.BlockSpec((BLK, H), lambda i: (i, 0)),
        pl.BlockSpec((H, H), lambda i: (0, 0)),
        pl.BlockSpec((H, H), lambda i: (0, 0)),
        pl.BlockSpec((1, H), lambda i: (0, 0)),
    ],
    out_specs=pl.BlockSpec((BLK, H), lambda i: (i, 0)),
    out_shape=jax.ShapeDtypeStruct((N, H), jnp.float32),
)


NP = N // 2
DP = 2 * H
BLKP = 1000


def _mlp_body(hr_ref, w1_ref, b1_ref, w2_ref, b2_ref, out_ref):
    hid = jnp.maximum(
        jnp.dot(hr_ref[...], w1_ref[...], preferred_element_type=jnp.float32)
        + b1_ref[...], 0.0)
    z = jnp.dot(hid, w2_ref[...], preferred_element_type=jnp.float32) + b2_ref[...]
    out_ref[...] = jax.nn.sigmoid(z)


_mlp = pl.pallas_call(
    _mlp_body,
    grid=(NP // BLKP,),
    in_specs=[
        pl.BlockSpec((BLKP, DP), lambda i: (i, 0)),
        pl.BlockSpec((DP, 3 * H), lambda i: (0, 0)),
        pl.BlockSpec((1, 3 * H), lambda i: (0, 0)),
        pl.BlockSpec((3 * H, OUT), lambda i: (0, 0)),
        pl.BlockSpec((1, OUT), lambda i: (0, 0)),
    ],
    out_specs=pl.BlockSpec((BLKP, OUT), lambda i: (i, 0)),
    out_shape=jax.ShapeDtypeStruct((NP, OUT), jnp.float32),
)


def kernel(x, edge_index, batch, W_rel0, b_rel0, W_root0, W_rel1, b_rel1,
           W_root1, mlp_W1, mlp_b1, mlp_W2, mlp_b2):
    src = edge_index[0]
    dst = edge_index[1]

    parts0 = _segsum(x, src, dst)
    h1 = _layer(parts0, x, W_rel0, W_root0, b_rel0.reshape(1, H))
    parts1 = _segsum(h1, src, dst)
    h2 = _layer(parts1, h1, W_rel1, W_root1, b_rel1.reshape(1, H))

    hr = h2.reshape(NP, DP)
    out = _mlp(hr, mlp_W1, mlp_b1.reshape(1, 3 * H), mlp_W2,
               mlp_b2.reshape(1, OUT))
    return (out, hr, hr[:, :H], hr[:, H:])
